# Initial kernel scaffold; baseline (speedup 1.0000x reference)
import jax
import jax.numpy as jnp
from jax.experimental import pallas as pl


def kernel(weight, u, k):
    raise NotImplementedError("write your pallas kernel here")


# speedup vs baseline: 2.9295x; 2.9295x over previous
import functools

import jax
import jax.numpy as jnp
from jax import lax
from jax.experimental import pallas as pl
from jax.experimental.pallas import tpu as pltpu
from jax.experimental.pallas import tpu_sc as plsc

K_ROWS = 100000
NZ = 64
PADW = 128

_BK = 10000
_NB = K_ROWS // _BK


def _sigma_body(u_ref, w_ref, o_ref, t_acc, g_acc):
    i = pl.program_id(0)

    @pl.when(i == 0)
    def _init():
        t_acc[...] = jnp.zeros_like(t_acc)
        g_acc[...] = jnp.zeros_like(g_acc)

    wb = w_ref[...]
    ub = u_ref[...]
    t_acc[...] += jnp.sum(wb * ub, axis=0, keepdims=True)
    g_acc[...] += lax.dot_general(
        wb, wb, (((0,), (0,)), ((), ())),
        preferred_element_type=jnp.float32,
        precision=lax.Precision.HIGHEST)

    @pl.when(i == _NB - 1)
    def _fin():
        t = t_acc[...]
        g = g_acc[...]
        v = t / (jnp.sqrt(jnp.sum(t * t)) + 1e-12)
        gv = lax.dot_general(
            v, g, (((1,), (0,)), ((), ())),
            preferred_element_type=jnp.float32,
            precision=lax.Precision.HIGHEST)
        sig2 = jnp.sum(gv * v[0, :])
        o_ref[0, 0] = 1.0 / jnp.sqrt(sig2)


def _recip_sigma(weight, u):
    return pl.pallas_call(
        _sigma_body,
        grid=(_NB,),
        in_specs=[
            pl.BlockSpec((_BK, 1), lambda i: (i, 0)),
            pl.BlockSpec((_BK, NZ), lambda i: (i, 0)),
        ],
        out_specs=pl.BlockSpec(memory_space=pltpu.SMEM),
        out_shape=jax.ShapeDtypeStruct((1, 1), jnp.float32),
        scratch_shapes=[
            pltpu.VMEM((1, NZ), jnp.float32),
            pltpu.VMEM((NZ, NZ), jnp.float32),
        ],
        compiler_params=pltpu.CompilerParams(
            dimension_semantics=("arbitrary",)),
    )(u.reshape(K_ROWS, 1), weight)


def _pad_body(r_ref, w_ref, o_ref):
    r = r_ref[0, 0]
    wb = w_ref[...]
    o_ref[...] = jnp.concatenate(
        [wb * r, jnp.zeros_like(wb)], axis=1)


def _scaled_pad(weight, recip):
    return pl.pallas_call(
        _pad_body,
        grid=(_NB,),
        in_specs=[
            pl.BlockSpec(memory_space=pltpu.SMEM),
            pl.BlockSpec((_BK, NZ), lambda i: (i, 0)),
        ],
        out_specs=pl.BlockSpec((_BK, PADW), lambda i: (i, 0)),
        out_shape=jax.ShapeDtypeStruct((K_ROWS, PADW), jnp.float32),
    )(recip, weight)


_NC = 2
_NS = 16
_NW = _NC * _NS
_B_TOTAL = 4096 * 50
_PER_W = _B_TOTAL // _NW
_RCH = 128
_NG = 5
_R = _RCH * _NG
_CH = _PER_W // _R
_IROWS_W = _PER_W // _RCH


def _gather_body(wpad_hbm, idx_hbm, out_hbm, idx_v, rows_v, sem):
    c = lax.axis_index("c")
    s = lax.axis_index("s")
    wid = s * _NC + c
    rbase = wid * _PER_W

    pltpu.sync_copy(idx_hbm.at[wid], idx_v)

    def chunk(i, carry):
        roff = rbase + i * _R
        cps = [
            pltpu.async_copy(
                wpad_hbm.at[idx_v.at[i * _NG + j]],
                rows_v.at[pl.ds(j * _RCH, _RCH)],
                sem)
            for j in range(_NG)
        ]
        for cp in cps:
            cp.wait()
        pltpu.sync_copy(rows_v, out_hbm.at[pl.ds(roff, _R)])
        return carry

    lax.fori_loop(0, _CH, chunk, 0)


def _gather(wpad, idx2d):
    mesh = plsc.VectorSubcoreMesh(
        core_axis_name="c", subcore_axis_name="s",
        num_cores=_NC, num_subcores=_NS)
    fn = pl.kernel(
        _gather_body,
        out_type=jax.ShapeDtypeStruct((_B_TOTAL, PADW), jnp.float32),
        mesh=mesh,
        scratch_types=[
            pltpu.VMEM((_IROWS_W, _RCH), jnp.int32),
            pltpu.VMEM((_R, PADW), jnp.float32),
            pltpu.SemaphoreType.DMA,
        ],
    )
    return fn(wpad, idx2d)


def kernel(weight, u, k):
    recip = _recip_sigma(weight, u)
    wpad = _scaled_pad(weight, recip)
    idx2d = k.reshape(_NW, _IROWS_W, _RCH).astype(jnp.int32)
    out = _gather(wpad, idx2d)
    return out[:, :NZ].reshape(k.shape[0], k.shape[1], NZ)
